# Initial kernel scaffold; baseline (speedup 1.0000x reference)
import jax
import jax.numpy as jnp
from jax.experimental import pallas as pl


def kernel(hidden_states, gate_weight, shared_w, shared_b, expert_u, expert_svh, expert_b):
    raise NotImplementedError("write your pallas kernel here")


# speedup vs baseline: 5.0080x; 5.0080x over previous
import functools

import jax
import jax.numpy as jnp
from jax.experimental import pallas as pl

N_EXP = 8
R_ROUTER = 8
R_EXP = 32
D_IN = 2048
D_OUT = 2048


def _smile_block(x_ref, gt_ref, swt_ref, svht_ref, ucat_ref, eb_ref, sb_ref,
                 out_ref, *, bn):
    x = x_ref[...]

    g = jnp.dot(x, gt_ref[...], preferred_element_type=jnp.float32)
    g2 = g * g
    lane = jax.lax.broadcasted_iota(jnp.int32, (N_EXP * R_ROUTER, N_EXP), 0)
    col = jax.lax.broadcasted_iota(jnp.int32, (N_EXP * R_ROUTER, N_EXP), 1)
    sel = (lane // R_ROUTER == col).astype(jnp.float32)
    logits = jnp.sqrt(jnp.dot(g2, sel, preferred_element_type=jnp.float32))

    iota = jax.lax.broadcasted_iota(jnp.int32, (bn, N_EXP), 1)
    m1 = jnp.max(logits, axis=1, keepdims=True)
    i1 = jnp.min(jnp.where(logits == m1, iota, N_EXP), axis=1, keepdims=True)
    sel1 = iota == i1
    l2 = jnp.where(sel1, -1.0, logits)
    m2 = jnp.max(l2, axis=1, keepdims=True)
    i2 = jnp.min(jnp.where(l2 == m2, iota, N_EXP), axis=1, keepdims=True)
    mask = sel1 | (iota == i2)
    ex = jnp.exp(logits - m1)
    exm = jnp.where(mask, ex, 0.0)
    w = exm / jnp.sum(exm, axis=1, keepdims=True)

    h = jnp.dot(x, svht_ref[...], preferred_element_type=jnp.float32)
    erow = jax.lax.broadcasted_iota(jnp.int32, (N_EXP, N_EXP * R_EXP), 0)
    ecol = jax.lax.broadcasted_iota(jnp.int32, (N_EXP, N_EXP * R_EXP), 1)
    expand = (ecol // R_EXP == erow).astype(jnp.float32)
    w_rep = jnp.dot(w, expand, preferred_element_type=jnp.float32)
    hw = h * w_rep

    out = jnp.dot(x, swt_ref[...], preferred_element_type=jnp.float32)
    out += jnp.dot(hw, ucat_ref[...], preferred_element_type=jnp.float32)
    out += jnp.dot(w, eb_ref[...], preferred_element_type=jnp.float32)
    out += sb_ref[...]
    out_ref[...] = out


@functools.partial(jax.jit, static_argnames=())
def kernel(hidden_states, gate_weight, shared_w, shared_b, expert_u,
           expert_svh, expert_b):
    b, s, d = hidden_states.shape
    n_tok = b * s
    x = hidden_states.reshape(n_tok, d)

    gate_t = gate_weight.T
    shared_t = shared_w.T
    svh_t = expert_svh.reshape(N_EXP * R_EXP, D_IN).T
    u_cat = expert_u.transpose(0, 2, 1).reshape(N_EXP * R_EXP, D_OUT)
    sb2 = shared_b.reshape(1, D_OUT)

    bn = 512
    grid = (n_tok // bn,)

    out = pl.pallas_call(
        functools.partial(_smile_block, bn=bn),
        grid=grid,
        in_specs=[
            pl.BlockSpec((bn, d), lambda i: (i, 0)),
            pl.BlockSpec((d, N_EXP * R_ROUTER), lambda i: (0, 0)),
            pl.BlockSpec((d, D_OUT), lambda i: (0, 0)),
            pl.BlockSpec((d, N_EXP * R_EXP), lambda i: (0, 0)),
            pl.BlockSpec((N_EXP * R_EXP, D_OUT), lambda i: (0, 0)),
            pl.BlockSpec((N_EXP, D_OUT), lambda i: (0, 0)),
            pl.BlockSpec((1, D_OUT), lambda i: (0, 0)),
        ],
        out_specs=pl.BlockSpec((bn, D_OUT), lambda i: (i, 0)),
        out_shape=jax.ShapeDtypeStruct((n_tok, D_OUT), jnp.float32),
    )(x, gate_t, shared_t, svh_t, u_cat, expert_b, sb2)

    return out.reshape(b, s, D_OUT)


# speedup vs baseline: 5.4313x; 1.0845x over previous
import functools

import jax
import jax.numpy as jnp
from jax.experimental import pallas as pl

N_EXP = 8
R_ROUTER = 8
R_EXP = 32
D_IN = 2048
D_OUT = 2048


def _smile_block(x_ref, gt_ref, swt_ref, svht_ref, ucat_ref, eb_ref, sb_ref,
                 out_ref, *, bn):
    x = x_ref[...]

    g = jnp.dot(x, gt_ref[...], preferred_element_type=jnp.float32)
    g2 = g * g
    lane = jax.lax.broadcasted_iota(jnp.int32, (N_EXP * R_ROUTER, N_EXP), 0)
    col = jax.lax.broadcasted_iota(jnp.int32, (N_EXP * R_ROUTER, N_EXP), 1)
    sel = (lane // R_ROUTER == col).astype(jnp.float32)
    logits = jnp.sqrt(jnp.dot(g2, sel, preferred_element_type=jnp.float32))

    iota = jax.lax.broadcasted_iota(jnp.int32, (bn, N_EXP), 1)
    m1 = jnp.max(logits, axis=1, keepdims=True)
    i1 = jnp.min(jnp.where(logits == m1, iota, N_EXP), axis=1, keepdims=True)
    sel1 = iota == i1
    l2 = jnp.where(sel1, -1.0, logits)
    m2 = jnp.max(l2, axis=1, keepdims=True)
    i2 = jnp.min(jnp.where(l2 == m2, iota, N_EXP), axis=1, keepdims=True)
    mask = sel1 | (iota == i2)
    ex = jnp.exp(logits - m1)
    exm = jnp.where(mask, ex, 0.0)
    w = exm / jnp.sum(exm, axis=1, keepdims=True)

    xb = x.astype(jnp.bfloat16)
    h = jnp.dot(xb, svht_ref[...], preferred_element_type=jnp.float32)
    erow = jax.lax.broadcasted_iota(jnp.int32, (N_EXP, N_EXP * R_EXP), 0)
    ecol = jax.lax.broadcasted_iota(jnp.int32, (N_EXP, N_EXP * R_EXP), 1)
    expand = (ecol // R_EXP == erow).astype(jnp.float32)
    w_rep = jnp.dot(w, expand, preferred_element_type=jnp.float32)
    hw = (h * w_rep).astype(jnp.bfloat16)

    out = jnp.dot(xb, swt_ref[...], preferred_element_type=jnp.float32)
    out += jnp.dot(hw, ucat_ref[...], preferred_element_type=jnp.float32)
    out += jnp.dot(w, eb_ref[...], preferred_element_type=jnp.float32)
    out += sb_ref[...]
    out_ref[...] = out


@functools.partial(jax.jit, static_argnames=())
def kernel(hidden_states, gate_weight, shared_w, shared_b, expert_u,
           expert_svh, expert_b):
    b, s, d = hidden_states.shape
    n_tok = b * s
    x = hidden_states.reshape(n_tok, d)

    gate_t = gate_weight.T
    shared_t = shared_w.T.astype(jnp.bfloat16)
    svh_t = expert_svh.reshape(N_EXP * R_EXP, D_IN).T.astype(jnp.bfloat16)
    u_cat = expert_u.transpose(0, 2, 1).reshape(
        N_EXP * R_EXP, D_OUT).astype(jnp.bfloat16)
    sb2 = shared_b.reshape(1, D_OUT)

    bn = 512
    grid = (n_tok // bn,)

    out = pl.pallas_call(
        functools.partial(_smile_block, bn=bn),
        grid=grid,
        in_specs=[
            pl.BlockSpec((bn, d), lambda i: (i, 0)),
            pl.BlockSpec((d, N_EXP * R_ROUTER), lambda i: (0, 0)),
            pl.BlockSpec((d, D_OUT), lambda i: (0, 0)),
            pl.BlockSpec((d, N_EXP * R_EXP), lambda i: (0, 0)),
            pl.BlockSpec((N_EXP * R_EXP, D_OUT), lambda i: (0, 0)),
            pl.BlockSpec((N_EXP, D_OUT), lambda i: (0, 0)),
            pl.BlockSpec((1, D_OUT), lambda i: (0, 0)),
        ],
        out_specs=pl.BlockSpec((bn, D_OUT), lambda i: (i, 0)),
        out_shape=jax.ShapeDtypeStruct((n_tok, D_OUT), jnp.float32),
    )(x, gate_t, shared_t, svh_t, u_cat, expert_b, sb2)

    return out.reshape(b, s, D_OUT)


# speedup vs baseline: 5.4369x; 1.0010x over previous
import functools

import jax
import jax.numpy as jnp
from jax.experimental import pallas as pl

N_EXP = 8
R_ROUTER = 8
R_EXP = 32
D_IN = 2048
D_OUT = 2048


def _smile_block(x_ref, gt_ref, swt_ref, svht_ref, ucat_ref, eb_ref, sb_ref,
                 out_ref, *, bn):
    x = x_ref[...]

    g = jnp.dot(x, gt_ref[...], preferred_element_type=jnp.float32)
    g2 = g * g
    lane = jax.lax.broadcasted_iota(jnp.int32, (N_EXP * R_ROUTER, N_EXP), 0)
    col = jax.lax.broadcasted_iota(jnp.int32, (N_EXP * R_ROUTER, N_EXP), 1)
    sel = (lane // R_ROUTER == col).astype(jnp.float32)
    logits = jnp.sqrt(jnp.dot(g2, sel, preferred_element_type=jnp.float32))

    iota = jax.lax.broadcasted_iota(jnp.int32, (bn, N_EXP), 1)
    m1 = jnp.max(logits, axis=1, keepdims=True)
    i1 = jnp.min(jnp.where(logits == m1, iota, N_EXP), axis=1, keepdims=True)
    sel1 = iota == i1
    l2 = jnp.where(sel1, -1.0, logits)
    m2 = jnp.max(l2, axis=1, keepdims=True)
    i2 = jnp.min(jnp.where(l2 == m2, iota, N_EXP), axis=1, keepdims=True)
    mask = sel1 | (iota == i2)
    ex = jnp.exp(logits - m1)
    exm = jnp.where(mask, ex, 0.0)
    w = exm / jnp.sum(exm, axis=1, keepdims=True)

    xb = x.astype(jnp.bfloat16)
    h = jnp.dot(xb, svht_ref[...], preferred_element_type=jnp.float32)
    erow = jax.lax.broadcasted_iota(jnp.int32, (N_EXP, N_EXP * R_EXP), 0)
    ecol = jax.lax.broadcasted_iota(jnp.int32, (N_EXP, N_EXP * R_EXP), 1)
    expand = (ecol // R_EXP == erow).astype(jnp.float32)
    w_rep = jnp.dot(w, expand, preferred_element_type=jnp.float32)
    hw = (h * w_rep).astype(jnp.bfloat16)

    out = jnp.dot(xb, swt_ref[...], preferred_element_type=jnp.float32)
    out += jnp.dot(hw, ucat_ref[...], preferred_element_type=jnp.float32)
    out += jnp.dot(w, eb_ref[...], preferred_element_type=jnp.float32)
    out += sb_ref[...]
    out_ref[...] = out


@functools.partial(jax.jit, static_argnames=())
def kernel(hidden_states, gate_weight, shared_w, shared_b, expert_u,
           expert_svh, expert_b):
    b, s, d = hidden_states.shape
    n_tok = b * s
    x = hidden_states.reshape(n_tok, d)

    gate_t = gate_weight.T
    shared_t = shared_w.T.astype(jnp.bfloat16)
    svh_t = expert_svh.reshape(N_EXP * R_EXP, D_IN).T.astype(jnp.bfloat16)
    u_cat = expert_u.transpose(0, 2, 1).reshape(
        N_EXP * R_EXP, D_OUT).astype(jnp.bfloat16)
    sb2 = shared_b.reshape(1, D_OUT)

    bn = 1024
    grid = (n_tok // bn,)

    out = pl.pallas_call(
        functools.partial(_smile_block, bn=bn),
        grid=grid,
        in_specs=[
            pl.BlockSpec((bn, d), lambda i: (i, 0)),
            pl.BlockSpec((d, N_EXP * R_ROUTER), lambda i: (0, 0)),
            pl.BlockSpec((d, D_OUT), lambda i: (0, 0)),
            pl.BlockSpec((d, N_EXP * R_EXP), lambda i: (0, 0)),
            pl.BlockSpec((N_EXP * R_EXP, D_OUT), lambda i: (0, 0)),
            pl.BlockSpec((N_EXP, D_OUT), lambda i: (0, 0)),
            pl.BlockSpec((1, D_OUT), lambda i: (0, 0)),
        ],
        out_specs=pl.BlockSpec((bn, D_OUT), lambda i: (i, 0)),
        out_shape=jax.ShapeDtypeStruct((n_tok, D_OUT), jnp.float32),
    )(x, gate_t, shared_t, svh_t, u_cat, expert_b, sb2)

    return out.reshape(b, s, D_OUT)


# speedup vs baseline: 5.9435x; 1.0932x over previous
import functools

import jax
import jax.numpy as jnp
from jax.experimental import pallas as pl
from jax.experimental.pallas import tpu as pltpu

N_EXP = 8
R_ROUTER = 8
R_EXP = 32
D_IN = 2048
D_OUT = 2048

_TDIMS = (((1,), (1,)), ((), ()))


def _smile_block(x_ref, gw_ref, sw_ref, svh_ref, ucat_ref, eb_ref, sb_ref,
                 out_ref, swb_ref, *, bn):
    @pl.when(pl.program_id(0) == 0)
    def _():
        swb_ref[...] = sw_ref[...].astype(jnp.bfloat16)

    x = x_ref[...]

    g = jax.lax.dot_general(x, gw_ref[...], _TDIMS,
                            preferred_element_type=jnp.float32)
    g2 = g * g
    lane = jax.lax.broadcasted_iota(jnp.int32, (N_EXP * R_ROUTER, N_EXP), 0)
    col = jax.lax.broadcasted_iota(jnp.int32, (N_EXP * R_ROUTER, N_EXP), 1)
    sel = (lane // R_ROUTER == col).astype(jnp.float32)
    logits = jnp.sqrt(jnp.dot(g2, sel, preferred_element_type=jnp.float32))

    iota = jax.lax.broadcasted_iota(jnp.int32, (bn, N_EXP), 1)
    m1 = jnp.max(logits, axis=1, keepdims=True)
    i1 = jnp.min(jnp.where(logits == m1, iota, N_EXP), axis=1, keepdims=True)
    sel1 = iota == i1
    l2 = jnp.where(sel1, -1.0, logits)
    m2 = jnp.max(l2, axis=1, keepdims=True)
    i2 = jnp.min(jnp.where(l2 == m2, iota, N_EXP), axis=1, keepdims=True)
    mask = sel1 | (iota == i2)
    ex = jnp.exp(logits - m1)
    exm = jnp.where(mask, ex, 0.0)
    w = exm / jnp.sum(exm, axis=1, keepdims=True)

    xb = x.astype(jnp.bfloat16)
    h = jax.lax.dot_general(xb, svh_ref[...], _TDIMS,
                            preferred_element_type=jnp.float32)
    erow = jax.lax.broadcasted_iota(jnp.int32, (N_EXP, N_EXP * R_EXP), 0)
    ecol = jax.lax.broadcasted_iota(jnp.int32, (N_EXP, N_EXP * R_EXP), 1)
    expand = (ecol // R_EXP == erow).astype(jnp.float32)
    w_rep = jnp.dot(w, expand, preferred_element_type=jnp.float32)
    hw = (h * w_rep).astype(jnp.bfloat16)

    out = jax.lax.dot_general(xb, swb_ref[...], _TDIMS,
                              preferred_element_type=jnp.float32)
    out += jnp.dot(hw, ucat_ref[...], preferred_element_type=jnp.float32)
    out += jnp.dot(w, eb_ref[...], preferred_element_type=jnp.float32)
    out += sb_ref[...]
    out_ref[...] = out


def kernel(hidden_states, gate_weight, shared_w, shared_b, expert_u,
           expert_svh, expert_b):
    b, s, d = hidden_states.shape
    n_tok = b * s
    x = hidden_states.reshape(n_tok, d)

    svh_b = expert_svh.reshape(N_EXP * R_EXP, D_IN).astype(jnp.bfloat16)
    u_cat = expert_u.transpose(0, 2, 1).reshape(
        N_EXP * R_EXP, D_OUT).astype(jnp.bfloat16)
    sb2 = shared_b.reshape(1, D_OUT)

    bn = 512
    grid = (n_tok // bn,)

    out = pl.pallas_call(
        functools.partial(_smile_block, bn=bn),
        grid=grid,
        in_specs=[
            pl.BlockSpec((bn, d), lambda i: (i, 0)),
            pl.BlockSpec((N_EXP * R_ROUTER, d), lambda i: (0, 0)),
            pl.BlockSpec((D_OUT, d), lambda i: (0, 0)),
            pl.BlockSpec((N_EXP * R_EXP, d), lambda i: (0, 0)),
            pl.BlockSpec((N_EXP * R_EXP, D_OUT), lambda i: (0, 0)),
            pl.BlockSpec((N_EXP, D_OUT), lambda i: (0, 0)),
            pl.BlockSpec((1, D_OUT), lambda i: (0, 0)),
        ],
        out_specs=pl.BlockSpec((bn, D_OUT), lambda i: (i, 0)),
        out_shape=jax.ShapeDtypeStruct((n_tok, D_OUT), jnp.float32),
        scratch_shapes=[pltpu.VMEM((D_OUT, D_IN), jnp.bfloat16)],
    )(x, gate_weight, shared_w, svh_b, u_cat, expert_b, sb2)

    return out.reshape(b, s, D_OUT)


# speedup vs baseline: 5.9514x; 1.0013x over previous
import functools

import jax
import jax.numpy as jnp
from jax.experimental import pallas as pl
from jax.experimental.pallas import tpu as pltpu

N_EXP = 8
R_ROUTER = 8
R_EXP = 32
D_IN = 2048
D_OUT = 2048

_TDIMS = (((1,), (1,)), ((), ()))


def _smile_block(x_ref, gw_ref, sw_ref, svh_ref, ucat_ref, eb_ref, sb_ref,
                 out_ref, swb_ref, *, bn):
    @pl.when(pl.program_id(0) == 0)
    def _():
        swb_ref[...] = sw_ref[...].astype(jnp.bfloat16)

    x = x_ref[...]

    g = jax.lax.dot_general(x, gw_ref[...], _TDIMS,
                            preferred_element_type=jnp.float32)
    g2 = g * g
    lane = jax.lax.broadcasted_iota(jnp.int32, (N_EXP * R_ROUTER, N_EXP), 0)
    col = jax.lax.broadcasted_iota(jnp.int32, (N_EXP * R_ROUTER, N_EXP), 1)
    sel = (lane // R_ROUTER == col).astype(jnp.float32)
    logits = jnp.sqrt(jnp.dot(g2, sel, preferred_element_type=jnp.float32))

    iota = jax.lax.broadcasted_iota(jnp.int32, (bn, N_EXP), 1)
    m1 = jnp.max(logits, axis=1, keepdims=True)
    i1 = jnp.min(jnp.where(logits == m1, iota, N_EXP), axis=1, keepdims=True)
    sel1 = iota == i1
    l2 = jnp.where(sel1, -1.0, logits)
    m2 = jnp.max(l2, axis=1, keepdims=True)
    i2 = jnp.min(jnp.where(l2 == m2, iota, N_EXP), axis=1, keepdims=True)
    mask = sel1 | (iota == i2)
    ex = jnp.exp(logits - m1)
    exm = jnp.where(mask, ex, 0.0)
    w = exm / jnp.sum(exm, axis=1, keepdims=True)

    xb = x.astype(jnp.bfloat16)
    h = jax.lax.dot_general(xb, svh_ref[...], _TDIMS,
                            preferred_element_type=jnp.float32)
    erow = jax.lax.broadcasted_iota(jnp.int32, (N_EXP, N_EXP * R_EXP), 0)
    ecol = jax.lax.broadcasted_iota(jnp.int32, (N_EXP, N_EXP * R_EXP), 1)
    expand = (ecol // R_EXP == erow).astype(jnp.float32)
    w_rep = jnp.dot(w, expand, preferred_element_type=jnp.float32)
    hw = (h * w_rep).astype(jnp.bfloat16)

    out = jax.lax.dot_general(xb, swb_ref[...], _TDIMS,
                              preferred_element_type=jnp.float32)
    out += jnp.dot(hw, ucat_ref[...], preferred_element_type=jnp.float32)
    out += jnp.dot(w, eb_ref[...], preferred_element_type=jnp.float32)
    out += sb_ref[...]
    out_ref[...] = out


def kernel(hidden_states, gate_weight, shared_w, shared_b, expert_u,
           expert_svh, expert_b):
    b, s, d = hidden_states.shape
    n_tok = b * s
    x = hidden_states.reshape(n_tok, d)

    svh_b = expert_svh.reshape(N_EXP * R_EXP, D_IN).astype(jnp.bfloat16)
    u_cat = expert_u.transpose(0, 2, 1).reshape(
        N_EXP * R_EXP, D_OUT).astype(jnp.bfloat16)
    sb2 = shared_b.reshape(1, D_OUT)

    bn = 512
    grid = (n_tok // bn,)

    out = pl.pallas_call(
        functools.partial(_smile_block, bn=bn),
        grid=grid,
        in_specs=[
            pl.BlockSpec((bn, d), lambda i: (i, 0)),
            pl.BlockSpec((N_EXP * R_ROUTER, d), lambda i: (0, 0)),
            pl.BlockSpec((D_OUT, d), lambda i: (0, 0)),
            pl.BlockSpec((N_EXP * R_EXP, d), lambda i: (0, 0)),
            pl.BlockSpec((N_EXP * R_EXP, D_OUT), lambda i: (0, 0)),
            pl.BlockSpec((N_EXP, D_OUT), lambda i: (0, 0)),
            pl.BlockSpec((1, D_OUT), lambda i: (0, 0)),
        ],
        out_specs=pl.BlockSpec((bn, D_OUT), lambda i: (i, 0)),
        out_shape=jax.ShapeDtypeStruct((n_tok, D_OUT), jnp.float32),
        scratch_shapes=[pltpu.VMEM((D_OUT, D_IN), jnp.bfloat16)],
        compiler_params=pltpu.CompilerParams(
            dimension_semantics=("arbitrary",),
        ),
    )(x, gate_weight, shared_w, svh_b, u_cat, expert_b, sb2)

    return out.reshape(b, s, D_OUT)


# speedup vs baseline: 6.0987x; 1.0247x over previous
import functools

import jax
import jax.numpy as jnp
from jax.experimental import pallas as pl
from jax.experimental.pallas import tpu as pltpu

N_EXP = 8
R_ROUTER = 8
R_EXP = 32
D_IN = 2048
D_OUT = 2048

_TDIMS = (((1,), (1,)), ((), ()))


def _smile_block(x_ref, gw_ref, sw_ref, svh_ref, ucat_ref, eb_ref, sb_ref,
                 out_ref, swb_ref, *, bn):
    @pl.when(pl.program_id(0) == 0)
    def _():
        swb_ref[...] = sw_ref[...].astype(jnp.bfloat16)

    x = x_ref[...]
    xb = x.astype(jnp.bfloat16)

    out = jax.lax.dot_general(xb, swb_ref[...], _TDIMS,
                              preferred_element_type=jnp.float32)
    h = jax.lax.dot_general(xb, svh_ref[...], _TDIMS,
                            preferred_element_type=jnp.float32)

    g = jax.lax.dot_general(x, gw_ref[...], _TDIMS,
                            preferred_element_type=jnp.float32)
    g2 = g * g
    lane = jax.lax.broadcasted_iota(jnp.int32, (N_EXP * R_ROUTER, N_EXP), 0)
    col = jax.lax.broadcasted_iota(jnp.int32, (N_EXP * R_ROUTER, N_EXP), 1)
    sel = (lane // R_ROUTER == col).astype(jnp.float32)
    logits = jnp.sqrt(jnp.dot(g2, sel, preferred_element_type=jnp.float32))

    iota = jax.lax.broadcasted_iota(jnp.int32, (bn, N_EXP), 1)
    m1 = jnp.max(logits, axis=1, keepdims=True)
    i1 = jnp.min(jnp.where(logits == m1, iota, N_EXP), axis=1, keepdims=True)
    sel1 = iota == i1
    l2 = jnp.where(sel1, -1.0, logits)
    m2 = jnp.max(l2, axis=1, keepdims=True)
    i2 = jnp.min(jnp.where(l2 == m2, iota, N_EXP), axis=1, keepdims=True)
    mask = sel1 | (iota == i2)
    ex = jnp.exp(logits - m1)
    exm = jnp.where(mask, ex, 0.0)
    w = exm / jnp.sum(exm, axis=1, keepdims=True)

    erow = jax.lax.broadcasted_iota(jnp.int32, (N_EXP, N_EXP * R_EXP), 0)
    ecol = jax.lax.broadcasted_iota(jnp.int32, (N_EXP, N_EXP * R_EXP), 1)
    expand = (ecol // R_EXP == erow).astype(jnp.float32)
    w_rep = jnp.dot(w, expand, preferred_element_type=jnp.float32)
    hw = (h * w_rep).astype(jnp.bfloat16)

    out += jnp.dot(hw, ucat_ref[...], preferred_element_type=jnp.float32)
    out += jnp.dot(w.astype(jnp.bfloat16), eb_ref[...],
                   preferred_element_type=jnp.float32)
    out += sb_ref[...]
    out_ref[...] = out


def kernel(hidden_states, gate_weight, shared_w, shared_b, expert_u,
           expert_svh, expert_b):
    b, s, d = hidden_states.shape
    n_tok = b * s
    x = hidden_states.reshape(n_tok, d)

    svh_b = expert_svh.reshape(N_EXP * R_EXP, D_IN).astype(jnp.bfloat16)
    u_cat = expert_u.transpose(0, 2, 1).reshape(
        N_EXP * R_EXP, D_OUT).astype(jnp.bfloat16)
    eb_b = expert_b.astype(jnp.bfloat16)
    sb2 = shared_b.reshape(1, D_OUT)

    bn = 512
    grid = (n_tok // bn,)

    out = pl.pallas_call(
        functools.partial(_smile_block, bn=bn),
        grid=grid,
        in_specs=[
            pl.BlockSpec((bn, d), lambda i: (i, 0)),
            pl.BlockSpec((N_EXP * R_ROUTER, d), lambda i: (0, 0)),
            pl.BlockSpec((D_OUT, d), lambda i: (0, 0)),
            pl.BlockSpec((N_EXP * R_EXP, d), lambda i: (0, 0)),
            pl.BlockSpec((N_EXP * R_EXP, D_OUT), lambda i: (0, 0)),
            pl.BlockSpec((N_EXP, D_OUT), lambda i: (0, 0)),
            pl.BlockSpec((1, D_OUT), lambda i: (0, 0)),
        ],
        out_specs=pl.BlockSpec((bn, D_OUT), lambda i: (i, 0)),
        out_shape=jax.ShapeDtypeStruct((n_tok, D_OUT), jnp.float32),
        scratch_shapes=[pltpu.VMEM((D_OUT, D_IN), jnp.bfloat16)],
        compiler_params=pltpu.CompilerParams(
            dimension_semantics=("arbitrary",),
        ),
    )(x, gate_weight, shared_w, svh_b, u_cat, eb_b, sb2)

    return out.reshape(b, s, D_OUT)


# speedup vs baseline: 6.2464x; 1.0242x over previous
import functools

import jax
import jax.numpy as jnp
from jax.experimental import pallas as pl
from jax.experimental.pallas import tpu as pltpu

N_EXP = 8
R_ROUTER = 8
R_EXP = 32
D_IN = 2048
D_OUT = 2048

_TDIMS = (((1,), (1,)), ((), ()))


def _smile_block(x_ref, gw_ref, sw_ref, svh_ref, ucat_ref, eb_ref, sb_ref,
                 out_ref, swb_ref, *, bn):
    @pl.when(pl.program_id(0) == 0)
    def _():
        swb_ref[...] = sw_ref[...].astype(jnp.bfloat16)

    x = x_ref[...]
    xb = x.astype(jnp.bfloat16)

    out = jax.lax.dot_general(xb, swb_ref[...], _TDIMS,
                              preferred_element_type=jnp.float32)
    h = jax.lax.dot_general(xb, svh_ref[...], _TDIMS,
                            preferred_element_type=jnp.float32)

    g = jax.lax.dot_general(x, gw_ref[...], _TDIMS,
                            preferred_element_type=jnp.float32)
    g2 = g * g
    lane = jax.lax.broadcasted_iota(jnp.int32, (N_EXP * R_ROUTER, N_EXP), 0)
    col = jax.lax.broadcasted_iota(jnp.int32, (N_EXP * R_ROUTER, N_EXP), 1)
    sel = (lane // R_ROUTER == col).astype(jnp.float32)
    logits = jnp.sqrt(jnp.dot(g2, sel, preferred_element_type=jnp.float32))

    iota = jax.lax.broadcasted_iota(
        jnp.int32, (bn, N_EXP), 1).astype(jnp.float32)
    m1 = jnp.max(logits, axis=1, keepdims=True)
    i1 = jnp.min(jnp.where(logits == m1, iota, 8.0), axis=1, keepdims=True)
    sel1 = iota == i1
    l2 = jnp.where(sel1, -1.0, logits)
    m2 = jnp.max(l2, axis=1, keepdims=True)
    i2 = jnp.min(jnp.where(l2 == m2, iota, 8.0), axis=1, keepdims=True)
    mask = sel1 | (iota == i2)
    ex = jnp.exp(logits - m1)
    exm = jnp.where(mask, ex, 0.0)
    w = exm / jnp.sum(exm, axis=1, keepdims=True)

    erow = jax.lax.broadcasted_iota(jnp.int32, (N_EXP, N_EXP * R_EXP), 0)
    ecol = jax.lax.broadcasted_iota(jnp.int32, (N_EXP, N_EXP * R_EXP), 1)
    expand = (ecol // R_EXP == erow).astype(jnp.float32)
    w_rep = jnp.dot(w, expand, preferred_element_type=jnp.float32)
    hw = (h * w_rep).astype(jnp.bfloat16)

    out += jnp.dot(hw, ucat_ref[...], preferred_element_type=jnp.float32)
    out += jnp.dot(w.astype(jnp.bfloat16), eb_ref[...],
                   preferred_element_type=jnp.float32)
    out += sb_ref[...]
    out_ref[...] = out


def kernel(hidden_states, gate_weight, shared_w, shared_b, expert_u,
           expert_svh, expert_b):
    b, s, d = hidden_states.shape
    n_tok = b * s
    x = hidden_states.reshape(n_tok, d)

    svh_b = expert_svh.reshape(N_EXP * R_EXP, D_IN).astype(jnp.bfloat16)
    u_cat = expert_u.transpose(0, 2, 1).reshape(
        N_EXP * R_EXP, D_OUT).astype(jnp.bfloat16)
    eb_b = expert_b.astype(jnp.bfloat16)
    sb2 = shared_b.reshape(1, D_OUT)

    bn = 512
    grid = (n_tok // bn,)

    out = pl.pallas_call(
        functools.partial(_smile_block, bn=bn),
        grid=grid,
        in_specs=[
            pl.BlockSpec((bn, d), lambda i: (i, 0)),
            pl.BlockSpec((N_EXP * R_ROUTER, d), lambda i: (0, 0)),
            pl.BlockSpec((D_OUT, d), lambda i: (0, 0)),
            pl.BlockSpec((N_EXP * R_EXP, d), lambda i: (0, 0)),
            pl.BlockSpec((N_EXP * R_EXP, D_OUT), lambda i: (0, 0)),
            pl.BlockSpec((N_EXP, D_OUT), lambda i: (0, 0)),
            pl.BlockSpec((1, D_OUT), lambda i: (0, 0)),
        ],
        out_specs=pl.BlockSpec((bn, D_OUT), lambda i: (i, 0)),
        out_shape=jax.ShapeDtypeStruct((n_tok, D_OUT), jnp.float32),
        scratch_shapes=[pltpu.VMEM((D_OUT, D_IN), jnp.bfloat16)],
        compiler_params=pltpu.CompilerParams(
            dimension_semantics=("arbitrary",),
        ),
    )(x, gate_weight, shared_w, svh_b, u_cat, eb_b, sb2)

    return out.reshape(b, s, D_OUT)
